# baseline (speedup 1.0000x reference)
import functools

import jax
import jax.numpy as jnp
from jax import lax
from jax.experimental import pallas as pl
from jax.experimental.pallas import tpu as pltpu
from jax.experimental.pallas import tpu_sc as plsc

N_CP = 20
CP_STEP = 40.0
D_PAD = 24

NC = 2
NS = 16
NW = NC * NS
LANES = 16
IDX_CHUNK = 128


@functools.partial(jax.jit, static_argnames=("b_per_w",))
def _trend_sc(t, idx, m_tab, k_tab, d_pad, b_per_w):
    B = t.shape[0]
    n_chunks = b_per_w // IDX_CHUNK
    n_groups = b_per_w // LANES
    mesh = plsc.VectorSubcoreMesh(core_axis_name="c", subcore_axis_name="s")

    @functools.partial(
        pl.kernel,
        mesh=mesh,
        compiler_params=pltpu.CompilerParams(
            needs_layout_passes=False, use_tc_tiling_on_sc=False
        ),
        out_type=jax.ShapeDtypeStruct((B,), jnp.float32),
        scratch_types=[
            [pltpu.VMEM((IDX_CHUNK,), jnp.int32)] * (b_per_w // IDX_CHUNK),
            pltpu.VMEM((b_per_w,), jnp.float32),
            pltpu.VMEM((b_per_w,), jnp.float32),
            pltpu.VMEM((b_per_w,), jnp.float32),
            pltpu.VMEM((b_per_w, D_PAD), jnp.float32),
            pltpu.VMEM((b_per_w,), jnp.float32),
            pltpu.SemaphoreType.DMA,
        ],
    )
    def sc_kernel(t_hbm, idx_hbm, m_hbm, k_hbm, d_hbm, out_hbm,
                  idx_vs, t_v, m_v, k_v, rows_v, out_v, sem):
        wid = lax.axis_index("s") * NC + lax.axis_index("c")
        base = wid * b_per_w

        for c in range(n_chunks):
            pltpu.sync_copy(
                idx_hbm.at[pl.ds(base + c * IDX_CHUNK, IDX_CHUNK)], idx_vs[c]
            )

        copies = []
        for c in range(n_chunks):
            sl = pl.ds(c * IDX_CHUNK, IDX_CHUNK)
            copies.append(
                pltpu.async_copy(d_hbm.at[idx_vs[c]], rows_v.at[sl], sem)
            )
            copies.append(pltpu.async_copy(m_hbm.at[idx_vs[c]], m_v.at[sl], sem))
            copies.append(pltpu.async_copy(k_hbm.at[idx_vs[c]], k_v.at[sl], sem))
        pltpu.sync_copy(t_hbm.at[pl.ds(base, b_per_w)], t_v)
        for cp in copies:
            cp.wait()

        lane = lax.iota(jnp.int32, LANES)

        def body(g, carry):
            gb = g * LANES
            tg = t_v[pl.ds(gb, LANES)]
            row_ix = gb + lane
            acc = m_v[pl.ds(gb, LANES)] + k_v[pl.ds(gb, LANES)] * tg
            for j in range(N_CP):
                col_ix = jnp.full((LANES,), j, jnp.int32)
                d = plsc.load_gather(rows_v, [row_ix, col_ix])
                sj = jnp.float32(CP_STEP * (j + 1))
                acc += jnp.where(tg > sj, d * (tg - sj), 0.0)
            out_v[pl.ds(gb, LANES)] = acc
            return carry

        lax.fori_loop(0, n_groups, body, 0)

        pltpu.sync_copy(out_v, out_hbm.at[pl.ds(base, b_per_w)])

    return sc_kernel(t, idx, m_tab, k_tab, d_pad)


def kernel(t, item_id, m_table, k_table, delta_table):
    B = t.shape[0]
    d_pad = jnp.pad(delta_table, ((0, 0), (0, D_PAD - N_CP)))
    out = _trend_sc(
        t.reshape(B),
        item_id.reshape(B),
        m_table.reshape(-1),
        k_table.reshape(-1),
        d_pad,
        b_per_w=B // NW,
    )
    return out.reshape(B, 1)


# speedup vs baseline: 2.3517x; 2.3517x over previous
import functools

import jax
import jax.numpy as jnp
from jax import lax
from jax.experimental import pallas as pl
from jax.experimental.pallas import tpu as pltpu
from jax.experimental.pallas import tpu_sc as plsc

N_CP = 20
CP_STEP = 40.0

NC = 2
NS = 16
NW = NC * NS
LANES = 16


@functools.partial(jax.jit, static_argnames=("b_per_w", "n_items"))
def _trend_sc(t, idx, m_tab, k_tab, d_flat, b_per_w, n_items):
    B = t.shape[0]
    n_groups = b_per_w // LANES
    mesh = plsc.VectorSubcoreMesh(core_axis_name="c", subcore_axis_name="s")

    @functools.partial(
        pl.kernel,
        mesh=mesh,
        compiler_params=pltpu.CompilerParams(
            needs_layout_passes=False, use_tc_tiling_on_sc=False
        ),
        out_type=jax.ShapeDtypeStruct((B,), jnp.float32),
        scratch_types=[
            pltpu.VMEM((b_per_w,), jnp.int32),
            [pltpu.VMEM((b_per_w,), jnp.int32)] * N_CP,
            pltpu.VMEM((N_CP * b_per_w,), jnp.float32),
            pltpu.VMEM((b_per_w,), jnp.float32),
            pltpu.VMEM((b_per_w,), jnp.float32),
            pltpu.VMEM((b_per_w,), jnp.float32),
            pltpu.VMEM((b_per_w,), jnp.float32),
            pltpu.SemaphoreType.DMA,
        ],
    )
    def sc_kernel(t_hbm, idx_hbm, m_hbm, k_hbm, d_hbm, out_hbm,
                  id_v, jx_vs, d_v, t_v, m_v, k_v, out_v, sem):
        wid = lax.axis_index("s") * NC + lax.axis_index("c")
        base = wid * b_per_w

        pltpu.sync_copy(idx_hbm.at[pl.ds(base, b_per_w)], id_v)
        copies = [
            pltpu.async_copy(m_hbm.at[id_v], m_v, sem),
            pltpu.async_copy(k_hbm.at[id_v], k_v, sem),
        ]
        for g in range(b_per_w // LANES):
            sl = pl.ds(g * LANES, LANES)
            ids = id_v[sl]
            for j in range(N_CP):
                jx_vs[j][sl] = ids + jnp.int32(n_items * j)
        for j in range(N_CP):
            copies.append(
                pltpu.async_copy(
                    d_hbm.at[jx_vs[j]], d_v.at[pl.ds(j * b_per_w, b_per_w)], sem
                )
            )
        pltpu.sync_copy(t_hbm.at[pl.ds(base, b_per_w)], t_v)
        for cp in copies:
            cp.wait()

        def body(g, carry):
            gb = g * LANES
            tg = t_v[pl.ds(gb, LANES)]
            acc = m_v[pl.ds(gb, LANES)] + k_v[pl.ds(gb, LANES)] * tg
            for j in range(N_CP):
                d = d_v[pl.ds(j * b_per_w + gb, LANES)]
                sj = jnp.float32(CP_STEP * (j + 1))
                acc += jnp.where(tg > sj, d * (tg - sj), 0.0)
            out_v[pl.ds(gb, LANES)] = acc
            return carry

        lax.fori_loop(0, n_groups, body, 0)

        pltpu.sync_copy(out_v, out_hbm.at[pl.ds(base, b_per_w)])

    return sc_kernel(t, idx, m_tab, k_tab, d_flat)


def kernel(t, item_id, m_table, k_table, delta_table):
    B = t.shape[0]
    n_items = delta_table.shape[0]
    out = _trend_sc(
        t.reshape(B),
        item_id.reshape(B),
        m_table.reshape(-1),
        k_table.reshape(-1),
        delta_table.T.reshape(-1),
        b_per_w=B // NW,
        n_items=n_items,
    )
    return out.reshape(B, 1)
